# baseline (speedup 1.0000x reference)
import functools

import jax
import jax.numpy as jnp
from jax.experimental import pallas as pl
from jax.experimental.pallas import tpu as pltpu

NBUF = 8
CR = 32


def _body(x_hbm, t_hbm, o_hbm, ibufs, obufs, stbl, tbl_v, in_sems, out_sems,
          tbl_sem, *, scale, nchunk):
    pltpu.make_async_copy(t_hbm, tbl_v, tbl_sem).start()
    pltpu.make_async_copy(t_hbm, tbl_v, tbl_sem).wait()
    stbl[...] = scale * tbl_v[...]

    for j in range(NBUF):
        pltpu.make_async_copy(
            x_hbm.at[pl.ds(j * CR, CR), :], ibufs.at[j], in_sems.at[j]
        ).start(priority=j % 2)

    for c in range(nchunk):
        slot = c % NBUF
        pltpu.make_async_copy(
            x_hbm.at[pl.ds(c * CR, CR), :], ibufs.at[slot], in_sems.at[slot]
        ).wait()
        if c >= NBUF:
            pltpu.make_async_copy(
                obufs.at[slot], o_hbm.at[pl.ds((c - NBUF) * CR, CR), :],
                out_sems.at[slot],
            ).wait()
        obufs[slot] = ibufs[slot] + stbl[...]
        pltpu.make_async_copy(
            obufs.at[slot], o_hbm.at[pl.ds(c * CR, CR), :], out_sems.at[slot]
        ).start(priority=c % 2)
        nxt = c + NBUF
        if nxt < nchunk:
            pltpu.make_async_copy(
                x_hbm.at[pl.ds(nxt * CR, CR), :], ibufs.at[slot],
                in_sems.at[slot],
            ).start(priority=nxt % 2)

    for c in range(nchunk - NBUF, nchunk):
        slot = c % NBUF
        pltpu.make_async_copy(
            obufs.at[slot], o_hbm.at[pl.ds(c * CR, CR), :], out_sems.at[slot]
        ).wait()


def kernel(inputs, lookup_table):
    B, T, D = inputs.shape
    scale = float(D) ** 0.5
    TD = T * D
    x = inputs.reshape(B, TD)
    tbl = lookup_table.reshape(1, TD)
    nchunk = B // CR

    out = pl.pallas_call(
        functools.partial(_body, scale=scale, nchunk=nchunk),
        in_specs=[
            pl.BlockSpec(memory_space=pl.ANY),
            pl.BlockSpec(memory_space=pl.ANY),
        ],
        out_specs=pl.BlockSpec(memory_space=pl.ANY),
        out_shape=jax.ShapeDtypeStruct((B, TD), jnp.float32),
        scratch_shapes=[
            pltpu.VMEM((NBUF, CR, TD), jnp.float32),
            pltpu.VMEM((NBUF, CR, TD), jnp.float32),
            pltpu.VMEM((1, TD), jnp.float32),
            pltpu.VMEM((1, TD), jnp.float32),
            pltpu.SemaphoreType.DMA((NBUF,)),
            pltpu.SemaphoreType.DMA((NBUF,)),
            pltpu.SemaphoreType.DMA,
        ],
    )(x, tbl)
    return out.reshape(B, T, D)


# speedup vs baseline: 3.4901x; 3.4901x over previous
import functools

import jax
import jax.numpy as jnp
from jax.experimental import pallas as pl
from jax.experimental.pallas import tpu as pltpu

RR = 256


def _body(x_ref, t_ref, o_ref, *, scale):
    o_ref[...] = x_ref[...] + scale * t_ref[...]


def kernel(inputs, lookup_table):
    B, T, D = inputs.shape
    scale = float(D) ** 0.5
    TD = T * D
    xt = jnp.transpose(inputs, (1, 2, 0)).reshape(TD, B)
    tbl = lookup_table.reshape(TD, 1)

    out = pl.pallas_call(
        functools.partial(_body, scale=scale),
        grid=(TD // RR,),
        in_specs=[
            pl.BlockSpec((RR, B), lambda i: (i, 0)),
            pl.BlockSpec((RR, 1), lambda i: (i, 0)),
        ],
        out_specs=pl.BlockSpec((RR, B), lambda i: (i, 0)),
        out_shape=jax.ShapeDtypeStruct((TD, B), jnp.float32),
    )(xt, tbl)
    return jnp.transpose(out.reshape(T, D, B), (2, 0, 1))


# speedup vs baseline: 3.5172x; 1.0078x over previous
import functools

import jax
import jax.numpy as jnp
from jax.experimental import pallas as pl
from jax.experimental.pallas import tpu as pltpu

RR = 512


def _body(x_ref, t_ref, o_ref, *, scale):
    o_ref[...] = x_ref[...] + scale * t_ref[...]


def kernel(inputs, lookup_table):
    B, T, D = inputs.shape
    scale = float(D) ** 0.5
    TD = T * D
    xt = jnp.transpose(inputs, (1, 2, 0)).reshape(TD, B)
    tbl = lookup_table.reshape(TD, 1)

    out = pl.pallas_call(
        functools.partial(_body, scale=scale),
        grid=(TD // RR,),
        in_specs=[
            pl.BlockSpec((RR, B), lambda i: (i, 0)),
            pl.BlockSpec((RR, 1), lambda i: (i, 0)),
        ],
        out_specs=pl.BlockSpec((RR, B), lambda i: (i, 0)),
        out_shape=jax.ShapeDtypeStruct((TD, B), jnp.float32),
    )(xt, tbl)
    return jnp.transpose(out.reshape(T, D, B), (2, 0, 1))
